# Initial kernel scaffold; baseline (speedup 1.0000x reference)
import jax
import jax.numpy as jnp
from jax.experimental import pallas as pl


def kernel(x, hidden_states, W, B, Wo, bo, edge_index):
    raise NotImplementedError("write your pallas kernel here")


# speedup vs baseline: 4.3224x; 4.3224x over previous
import functools

import jax
import jax.numpy as jnp
from jax import lax
from jax.experimental import pallas as pl
from jax.experimental.pallas import tpu as pltpu
from jax.experimental.pallas import tpu_sc as plsc

N = 10000
E = 320000
F = 128
NL = 2
NS_STEPS = 4

NC = 2
NSUB = 16
K = 80
EPT = E // NSUB
NCHUNK = EPT // K
NP = 10240
ROWS_PT = NP // NSUB

_mesh = plsc.VectorSubcoreMesh(
    core_axis_name="c", subcore_axis_name="s", num_cores=NC, num_subcores=NSUB)


@functools.partial(
    pl.kernel,
    out_type=pltpu.HBM((2 * NP, F), jnp.float32),
    mesh=_mesh,
    scratch_types=[
        pltpu.VMEM((EPT,), jnp.int32),
        pltpu.VMEM((K,), jnp.int32),
        pltpu.VMEM((K,), jnp.int32),
        pltpu.VMEM((K, F), jnp.float32),
        pltpu.VMEM((K, F), jnp.float32),
        pltpu.VMEM((K,), jnp.int32),
        pltpu.VMEM_SHARED((NP, F), jnp.float32),
        pltpu.SemaphoreType.DMA,
        pltpu.SemaphoreType.DMA,
    ],
)
def _agg(tbl_hbm, src_hbm, dst_hbm, out,
         src_v, dst0, dst1, rows0, rows1, idx_z, acc, sem0, sem1):
    c = lax.axis_index("c")
    s = lax.axis_index("s")

    e0 = s * EPT
    pltpu.sync_copy(src_hbm.at[pl.ds(e0, EPT)], src_v)
    coff = c * N

    def shift(i, carry):
        src_v[pl.ds(16 * i, 16)] = src_v[pl.ds(16 * i, 16)] + coff
        return carry

    lax.fori_loop(0, EPT // 16, shift, 0)

    zero16 = jnp.zeros((16,), jnp.float32)

    def zrow(r, carry):
        for k in range(F // 16):
            rows1[r, pl.ds(16 * k, 16)] = zero16
        return carry

    lax.fori_loop(0, K, zrow, 0)
    r0 = s * ROWS_PT
    iota16 = lax.iota(jnp.int32, 16)
    for q in range(ROWS_PT // K):
        for k in range(K // 16):
            idx_z[pl.ds(16 * k, 16)] = iota16 + (r0 + q * K + 16 * k)
        pltpu.sync_copy(rows1, acc.at[idx_z])
    plsc.subcore_barrier()

    def issue_gather(j, buf, sem):
        pltpu.async_copy(tbl_hbm.at[src_v.at[pl.ds(j * K, K)]], buf, sem)

    def wait_gather(buf, sem):
        pltpu.make_async_copy(tbl_hbm.at[src_v.at[pl.ds(0, K)]], buf, sem).wait()

    def load_dst(j, dbuf):
        pltpu.sync_copy(dst_hbm.at[pl.ds(e0 + j * K, K)], dbuf)

    issue_gather(0, rows0, sem0)
    issue_gather(1, rows1, sem1)
    load_dst(0, dst0)
    load_dst(1, dst1)

    def body(g, carry):
        j0 = 2 * g

        wait_gather(rows0, sem0)
        pltpu.sync_copy(rows0, acc.at[dst0], add=True)

        @pl.when(j0 + 2 < NCHUNK)
        def _():
            issue_gather(j0 + 2, rows0, sem0)
            load_dst(j0 + 2, dst0)

        wait_gather(rows1, sem1)
        pltpu.sync_copy(rows1, acc.at[dst1], add=True)

        @pl.when(j0 + 3 < NCHUNK)
        def _():
            issue_gather(j0 + 3, rows1, sem1)
            load_dst(j0 + 3, dst1)

        return carry

    lax.fori_loop(0, NCHUNK // 2, body, 0)
    plsc.subcore_barrier()

    ob = c * NP + r0
    for q in range(ROWS_PT // K):
        pltpu.sync_copy(acc.at[pl.ds(r0 + q * K, K)], rows0)
        pltpu.sync_copy(rows0, out.at[pl.ds(ob + q * K, K)])


_RB = 1000


def _cell_body(ax_ref, ah_ref, deg_ref, h_ref, w_ref, b_ref, out_ref):
    di = 1.0 / jnp.maximum(deg_ref[...], 1.0)
    m = jnp.concatenate([ax_ref[...] * di, ah_ref[...] * di], axis=1)
    pre = jnp.dot(m, w_ref[...], preferred_element_type=jnp.float32) + b_ref[...]
    u = jax.nn.sigmoid(pre[:, :F])
    cand = jnp.tanh(pre[:, F:])
    h = h_ref[...]
    out_ref[...] = u * h + (1.0 - u) * cand


_cell = pl.pallas_call(
    _cell_body,
    grid=(N // _RB,),
    in_specs=[
        pl.BlockSpec((_RB, F), lambda i: (i, 0)),
        pl.BlockSpec((_RB, F), lambda i: (i, 0)),
        pl.BlockSpec((_RB, 1), lambda i: (i, 0)),
        pl.BlockSpec((_RB, F), lambda i: (i, 0)),
        pl.BlockSpec((2 * F, 2 * F), lambda i: (0, 0)),
        pl.BlockSpec((1, 2 * F), lambda i: (0, 0)),
    ],
    out_specs=pl.BlockSpec((_RB, F), lambda i: (i, 0)),
    out_shape=jax.ShapeDtypeStruct((N, F), jnp.float32),
)


def _proj_body(y_ref, w_ref, b_ref, out_ref):
    out_ref[...] = (
        jnp.dot(y_ref[...], w_ref[...], preferred_element_type=jnp.float32)
        + b_ref[...])


_proj = pl.pallas_call(
    _proj_body,
    grid=(NS_STEPS * N // _RB,),
    in_specs=[
        pl.BlockSpec((_RB, F), lambda i: (i, 0)),
        pl.BlockSpec((F, F), lambda i: (0, 0)),
        pl.BlockSpec((1, F), lambda i: (0, 0)),
    ],
    out_specs=pl.BlockSpec((_RB, F), lambda i: (i, 0)),
    out_shape=jax.ShapeDtypeStruct((NS_STEPS * N, F), jnp.float32),
)


def kernel(x, hidden_states, W, B, Wo, bo, edge_index):
    src = edge_index[0]
    dst = edge_index[1]

    ones_tbl = jnp.ones((2 * N, F), jnp.float32)
    deg = _agg(ones_tbl, src, dst)[:N, :1]

    wcat, bcat = [], []
    for j in range(NL):
        wj = jnp.concatenate([
            jnp.concatenate([W[j, 2], W[j, 4]], axis=1),
            jnp.concatenate([W[j, 3], W[j, 5]], axis=1)], axis=0)
        bj = jnp.concatenate([B[j, 2] + B[j, 3], B[j, 4] + B[j, 5]])[None, :]
        wcat.append(wj)
        bcat.append(bj)

    hid = [hidden_states[j] for j in range(NL)]
    outs = []
    for i in range(NS_STEPS):
        inp = x[i]
        for j in range(NL):
            tbl = jnp.concatenate([inp, hid[j]], axis=0)
            agg = _agg(tbl, src, dst)
            inp = _cell(agg[:N], agg[NP:NP + N], deg, hid[j],
                        wcat[j], bcat[j])
            hid[j] = inp
        outs.append(inp)

    y = jnp.stack(outs).reshape(NS_STEPS * N, F)
    out = _proj(y, Wo, bo[None, :]).reshape(NS_STEPS, N, F)
    return (out, jnp.stack(hid))
